# initial kernel scaffold (unmeasured)
import jax
import jax.numpy as jnp
from jax import lax
from jax.experimental import pallas as pl
from jax.experimental.pallas import tpu as pltpu

B = 8
H = 8
D = 128
BS = 16
BLOCK = 32
T = BLOCK * BS

_MESH = pltpu.DeviceIdType.MESH
NEG_INF = -1e30


def _partial_body(q_ref, w_ref, k_ref, v_ref, m_ref, l_ref, acc_ref):
    pb = pl.program_id(0)

    @pl.when(pb == 0)
    def _():
        m_ref[...] = jnp.full_like(m_ref, NEG_INF)
        l_ref[...] = jnp.zeros_like(l_ref)
        acc_ref[...] = jnp.zeros_like(acc_ref)

    q = q_ref[...]
    k = k_ref[...].reshape(T, H, D)
    v = v_ref[...].reshape(T, H, D)
    w = w_ref[...]

    s = lax.dot_general(
        q, k,
        dimension_numbers=(((2,), (2,)), ((1,), (1,))),
        preferred_element_type=jnp.float32,
    )

    m_old = m_ref[...]
    m_new = jnp.maximum(m_old, jnp.max(s, axis=2))
    corr = jnp.exp(m_old - m_new)
    p = jnp.exp(s - m_new[:, :, None]) * w[None, :, :]
    l_ref[...] = l_ref[...] * corr + jnp.sum(p, axis=2)
    pv = lax.dot_general(
        p, v,
        dimension_numbers=(((2,), (0,)), ((0,), (1,))),
        preferred_element_type=jnp.float32,
    )
    acc_ref[...] = acc_ref[...] * corr[:, :, None] + pv
    m_ref[...] = m_new


def _combine_body(m_ref, l_ref, acc_ref, out_ref,
                  m_r, l_r, acc_r, send_sems, recv_sems):
    my_x = lax.axis_index("x")
    my_y = lax.axis_index("y")
    my_z = lax.axis_index("z")
    peer = (1 - my_x, my_y, my_z)

    barrier_sem = pltpu.get_barrier_semaphore()
    pl.semaphore_signal(barrier_sem, inc=1, device_id=peer,
                        device_id_type=_MESH)
    pl.semaphore_wait(barrier_sem, 1)

    copies = []
    for i, (src, dst) in enumerate(((m_ref, m_r), (l_ref, l_r),
                                    (acc_ref, acc_r))):
        c = pltpu.make_async_remote_copy(
            src_ref=src, dst_ref=dst,
            send_sem=send_sems.at[i], recv_sem=recv_sems.at[i],
            device_id=peer, device_id_type=_MESH,
        )
        c.start()
        copies.append(c)
    for c in copies:
        c.wait()

    m_a, m_b = m_ref[...], m_r[...]
    m = jnp.maximum(m_a, m_b)
    ea = jnp.exp(m_a - m)
    eb = jnp.exp(m_b - m)
    l = l_ref[...] * ea + l_r[...] * eb
    acc = acc_ref[...] * ea[:, :, None] + acc_r[...] * eb[:, :, None]
    o = acc / l[:, :, None]
    out_ref[...] = o.transpose(1, 0, 2)[:, None, :, :]


def kernel(Q, K, V, bt, lens):
    n_local = K.shape[0]
    nb = bt.shape[1]
    assert n_local % BLOCK == 0
    nsteps = n_local // BLOCK

    my_x = lax.axis_index("x")
    gid = my_x * n_local + jnp.arange(n_local, dtype=jnp.int32)
    valid = jnp.arange(nb, dtype=jnp.int32)[None, :] < lens[:, None]
    hits = (bt[:, :, None] == gid[None, None, :]) & valid[:, :, None]
    w_page = jnp.sum(hits.astype(jnp.float32), axis=1)
    w_tok = jnp.repeat(w_page, BS, axis=1)

    q = Q.reshape(B, H, D).astype(jnp.float32) * (D ** -0.5)

    m, l, acc = pl.pallas_call(
        _partial_body,
        grid=(nsteps,),
        in_specs=[
            pl.BlockSpec((B, H, D), lambda pb: (0, 0, 0)),
            pl.BlockSpec((B, T), lambda pb: (0, pb)),
            pl.BlockSpec((BLOCK, BS, H, D), lambda pb: (pb, 0, 0, 0)),
            pl.BlockSpec((BLOCK, BS, H, D), lambda pb: (pb, 0, 0, 0)),
        ],
        out_specs=[
            pl.BlockSpec((H, B), lambda pb: (0, 0)),
            pl.BlockSpec((H, B), lambda pb: (0, 0)),
            pl.BlockSpec((H, B, D), lambda pb: (0, 0, 0)),
        ],
        out_shape=[
            jax.ShapeDtypeStruct((H, B), jnp.float32),
            jax.ShapeDtypeStruct((H, B), jnp.float32),
            jax.ShapeDtypeStruct((H, B, D), jnp.float32),
        ],
    )(q, w_tok, K, V)

    out = pl.pallas_call(
        _combine_body,
        in_specs=[
            pl.BlockSpec(memory_space=pltpu.VMEM),
            pl.BlockSpec(memory_space=pltpu.VMEM),
            pl.BlockSpec(memory_space=pltpu.VMEM),
        ],
        out_specs=pl.BlockSpec(memory_space=pltpu.VMEM),
        out_shape=jax.ShapeDtypeStruct((B, 1, H, D), jnp.float32),
        scratch_shapes=[
            pltpu.VMEM((H, B), jnp.float32),
            pltpu.VMEM((H, B), jnp.float32),
            pltpu.VMEM((H, B, D), jnp.float32),
            pltpu.SemaphoreType.DMA((3,)),
            pltpu.SemaphoreType.DMA((3,)),
        ],
        compiler_params=pltpu.CompilerParams(collective_id=0),
    )(m, l, acc)
    return out


# baseline (device time: 92510 ns/iter reference)
import jax
import jax.numpy as jnp
from jax import lax
from jax.experimental import pallas as pl
from jax.experimental.pallas import tpu as pltpu

B = 8
H = 8
D = 128
BS = 16
HB = H * B
T = 512

_MESH = pltpu.DeviceIdType.MESH
NEG_INF = -1e30


def _partial_body(qbd_ref, w_ref, k_ref, v_ref, m_ref, l_ref, acc_ref):
    pb = pl.program_id(0)

    @pl.when(pb == 0)
    def _():
        m_ref[...] = jnp.full_like(m_ref, NEG_INF)
        l_ref[...] = jnp.zeros_like(l_ref)
        acc_ref[...] = jnp.zeros_like(acc_ref)

    k = k_ref[...]
    v = v_ref[...]
    w = w_ref[...]

    s = lax.dot_general(
        k, qbd_ref[...],
        dimension_numbers=(((1,), (0,)), ((), ())),
        preferred_element_type=jnp.float32,
    )

    m_old = m_ref[...]
    m_new = jnp.maximum(m_old, jnp.max(s, axis=0, keepdims=True))
    corr = jnp.exp(m_old - m_new)
    p = jnp.exp(s - m_new) * w
    l_ref[...] = l_ref[...] * corr + jnp.sum(p, axis=0, keepdims=True)

    pv_full = lax.dot_general(
        v, p,
        dimension_numbers=(((0,), (0,)), ((), ())),
        preferred_element_type=jnp.float32,
    )
    diag = jnp.concatenate(
        [pv_full[h * D:(h + 1) * D, h * B:(h + 1) * B] for h in range(H)],
        axis=1,
    )
    acc_ref[...] = acc_ref[...] * corr + diag
    m_ref[...] = m_new


def _combine_body(m_ref, l_ref, acc_ref, out_ref,
                  m_r, l_r, acc_r, send_sems, recv_sems):
    my_x = lax.axis_index("x")
    my_y = lax.axis_index("y")
    my_z = lax.axis_index("z")
    peer = (1 - my_x, my_y, my_z)

    barrier_sem = pltpu.get_barrier_semaphore()
    pl.semaphore_signal(barrier_sem, inc=1, device_id=peer,
                        device_id_type=_MESH)
    pl.semaphore_wait(barrier_sem, 1)

    copies = []
    for i, (src, dst) in enumerate(((m_ref, m_r), (l_ref, l_r),
                                    (acc_ref, acc_r))):
        c = pltpu.make_async_remote_copy(
            src_ref=src, dst_ref=dst,
            send_sem=send_sems.at[i], recv_sem=recv_sems.at[i],
            device_id=peer, device_id_type=_MESH,
        )
        c.start()
        copies.append(c)
    for c in copies:
        c.wait()

    m_a, m_b = m_ref[...], m_r[...]
    m = jnp.maximum(m_a, m_b)
    ea = jnp.exp(m_a - m)
    eb = jnp.exp(m_b - m)
    l = l_ref[...] * ea + l_r[...] * eb
    o = (acc_ref[...] * ea + acc_r[...] * eb) / l
    for h in range(H):
        out_ref[:, 0, h, :] = o[:, h * B:(h + 1) * B].T


def kernel(Q, K, V, bt, lens):
    n_local = K.shape[0]
    nb = bt.shape[1]
    n_tok = n_local * BS
    assert n_tok % T == 0
    nsteps = n_tok // T

    my_x = lax.axis_index("x")
    gid = my_x * n_local + jnp.arange(n_local, dtype=jnp.int32)
    valid = jnp.arange(nb, dtype=jnp.int32)[None, :] < lens[:, None]
    hits = (bt[:, :, None] == gid[None, None, :]) & valid[:, :, None]
    w_page = jnp.sum(hits.astype(jnp.float32), axis=1)
    w_tok = jnp.repeat(w_page, BS, axis=1)
    w2 = jnp.tile(w_tok.T, (1, H))

    q = Q.reshape(B, H, D).astype(jnp.float32) * (D ** -0.5)
    qbd = (jnp.eye(H, dtype=jnp.float32)[:, None, :, None]
           * q.transpose(1, 2, 0)[:, :, None, :]).reshape(H * D, HB)

    k2 = K.reshape(n_tok, H * D)
    v2 = V.reshape(n_tok, H * D)

    m, l, acc = pl.pallas_call(
        _partial_body,
        grid=(nsteps,),
        in_specs=[
            pl.BlockSpec((H * D, HB), lambda pb: (0, 0)),
            pl.BlockSpec((T, HB), lambda pb: (pb, 0)),
            pl.BlockSpec((T, H * D), lambda pb: (pb, 0)),
            pl.BlockSpec((T, H * D), lambda pb: (pb, 0)),
        ],
        out_specs=[
            pl.BlockSpec((1, HB), lambda pb: (0, 0)),
            pl.BlockSpec((1, HB), lambda pb: (0, 0)),
            pl.BlockSpec((D, HB), lambda pb: (0, 0)),
        ],
        out_shape=[
            jax.ShapeDtypeStruct((1, HB), jnp.float32),
            jax.ShapeDtypeStruct((1, HB), jnp.float32),
            jax.ShapeDtypeStruct((D, HB), jnp.float32),
        ],
    )(qbd, w2, k2, v2)

    out = pl.pallas_call(
        _combine_body,
        in_specs=[
            pl.BlockSpec(memory_space=pltpu.VMEM),
            pl.BlockSpec(memory_space=pltpu.VMEM),
            pl.BlockSpec(memory_space=pltpu.VMEM),
        ],
        out_specs=pl.BlockSpec(memory_space=pltpu.VMEM),
        out_shape=jax.ShapeDtypeStruct((B, 1, H, D), jnp.float32),
        scratch_shapes=[
            pltpu.VMEM((1, HB), jnp.float32),
            pltpu.VMEM((1, HB), jnp.float32),
            pltpu.VMEM((D, HB), jnp.float32),
            pltpu.SemaphoreType.DMA((3,)),
            pltpu.SemaphoreType.DMA((3,)),
        ],
        compiler_params=pltpu.CompilerParams(collective_id=0),
    )(m, l, acc)
    return out


# device time: 88471 ns/iter; 1.0457x vs baseline; 1.0457x over previous
import jax
import jax.numpy as jnp
from jax import lax
from jax.experimental import pallas as pl
from jax.experimental.pallas import tpu as pltpu

B = 8
H = 8
D = 128
BS = 16
HB = H * B
T = 512
PG = T // BS

_MESH = pltpu.DeviceIdType.MESH
NEG_INF = -1e30


def _partial_body(qbd_ref, bt_ref, lens_ref, k_ref, v_ref,
                  m_ref, l_ref, acc_ref):
    pb = pl.program_id(0)
    nb = bt_ref.shape[0]

    @pl.when(pb == 0)
    def _():
        m_ref[...] = jnp.full_like(m_ref, NEG_INF)
        l_ref[...] = jnp.zeros_like(l_ref)
        acc_ref[...] = jnp.zeros_like(acc_ref)

    my_x = lax.axis_index("x")
    n_local_pages = pl.num_programs(0) * PG
    base = my_x * n_local_pages + pb * PG
    bt = bt_ref[...]
    gid = base + lax.broadcasted_iota(jnp.int32, (nb, PG), 1)
    jio = lax.broadcasted_iota(jnp.int32, (nb, PG), 0)
    rows = []
    for b in range(B):
        hit = (bt[:, b:b + 1] == gid) & (jio < lens_ref[0, b])
        rows.append(jnp.sum(hit.astype(jnp.float32), axis=0, keepdims=True))
    wpg = jnp.concatenate(rows, axis=0)
    expand = (lax.broadcasted_iota(jnp.int32, (PG, T), 1) // BS
              == lax.broadcasted_iota(jnp.int32, (PG, T), 0))
    w8 = lax.dot_general(
        wpg, expand.astype(jnp.float32),
        dimension_numbers=(((1,), (0,)), ((), ())),
        preferred_element_type=jnp.float32,
    )
    w = jnp.concatenate([w8] * H, axis=0)

    s = lax.dot_general(
        qbd_ref[...], k_ref[...],
        dimension_numbers=(((0,), (1,)), ((), ())),
        preferred_element_type=jnp.float32,
    )

    m_old = m_ref[...]
    m_new = jnp.maximum(m_old, jnp.max(s, axis=1, keepdims=True))
    corr = jnp.exp(m_old - m_new)
    p = jnp.exp(s - m_new) * w
    l_ref[...] = l_ref[...] * corr + jnp.sum(p, axis=1, keepdims=True)

    pv_full = lax.dot_general(
        p, v_ref[...],
        dimension_numbers=(((1,), (0,)), ((), ())),
        preferred_element_type=jnp.float32,
    )
    diag = jnp.concatenate(
        [pv_full[h * B:(h + 1) * B, h * D:(h + 1) * D] for h in range(H)],
        axis=0,
    )
    acc_ref[...] = acc_ref[...] * corr + diag
    m_ref[...] = m_new


def _combine_body(m_ref, l_ref, acc_ref, out_ref,
                  m_r, l_r, acc_r, send_sems, recv_sems):
    my_x = lax.axis_index("x")
    my_y = lax.axis_index("y")
    my_z = lax.axis_index("z")
    peer = (1 - my_x, my_y, my_z)

    barrier_sem = pltpu.get_barrier_semaphore()
    pl.semaphore_signal(barrier_sem, inc=1, device_id=peer,
                        device_id_type=_MESH)
    pl.semaphore_wait(barrier_sem, 1)

    copies = []
    for i, (src, dst) in enumerate(((m_ref, m_r), (l_ref, l_r),
                                    (acc_ref, acc_r))):
        c = pltpu.make_async_remote_copy(
            src_ref=src, dst_ref=dst,
            send_sem=send_sems.at[i], recv_sem=recv_sems.at[i],
            device_id=peer, device_id_type=_MESH,
        )
        c.start()
        copies.append(c)
    for c in copies:
        c.wait()

    m_a, m_b = m_ref[...], m_r[...]
    m = jnp.maximum(m_a, m_b)
    ea = jnp.exp(m_a - m)
    eb = jnp.exp(m_b - m)
    l = l_ref[...] * ea + l_r[...] * eb
    o = (acc_ref[...] * ea + acc_r[...] * eb) / l
    for h in range(H):
        out_ref[:, 0, h, :] = o[h * B:(h + 1) * B, :]


def kernel(Q, K, V, bt, lens):
    n_local = K.shape[0]
    nb = bt.shape[1]
    n_tok = n_local * BS
    assert n_tok % T == 0
    nsteps = n_tok // T

    q = Q.reshape(B, H, D).astype(jnp.float32) * (D ** -0.5)
    qbd = (jnp.eye(H, dtype=jnp.float32)[:, None, :, None]
           * q.transpose(1, 2, 0)[:, :, None, :]).reshape(H * D, HB)

    bt_t = bt.T
    lens2 = lens.reshape(1, B)
    k2 = K.reshape(n_tok, H * D)
    v2 = V.reshape(n_tok, H * D)

    m, l, acc = pl.pallas_call(
        _partial_body,
        grid=(nsteps,),
        in_specs=[
            pl.BlockSpec((H * D, HB), lambda pb: (0, 0)),
            pl.BlockSpec((nb, B), lambda pb: (0, 0)),
            pl.BlockSpec(memory_space=pltpu.SMEM),
            pl.BlockSpec((T, H * D), lambda pb: (pb, 0)),
            pl.BlockSpec((T, H * D), lambda pb: (pb, 0)),
        ],
        out_specs=[
            pl.BlockSpec((HB, 1), lambda pb: (0, 0)),
            pl.BlockSpec((HB, 1), lambda pb: (0, 0)),
            pl.BlockSpec((HB, D), lambda pb: (0, 0)),
        ],
        out_shape=[
            jax.ShapeDtypeStruct((HB, 1), jnp.float32),
            jax.ShapeDtypeStruct((HB, 1), jnp.float32),
            jax.ShapeDtypeStruct((HB, D), jnp.float32),
        ],
    )(qbd, bt_t, lens2, k2, v2)

    out = pl.pallas_call(
        _combine_body,
        in_specs=[
            pl.BlockSpec(memory_space=pltpu.VMEM),
            pl.BlockSpec(memory_space=pltpu.VMEM),
            pl.BlockSpec(memory_space=pltpu.VMEM),
        ],
        out_specs=pl.BlockSpec(memory_space=pltpu.VMEM),
        out_shape=jax.ShapeDtypeStruct((B, 1, H, D), jnp.float32),
        scratch_shapes=[
            pltpu.VMEM((HB, 1), jnp.float32),
            pltpu.VMEM((HB, 1), jnp.float32),
            pltpu.VMEM((HB, D), jnp.float32),
            pltpu.SemaphoreType.DMA((3,)),
            pltpu.SemaphoreType.DMA((3,)),
        ],
        compiler_params=pltpu.CompilerParams(collective_id=0),
    )(m, l, acc)
    return out


# device time: 44111 ns/iter; 2.0972x vs baseline; 2.0056x over previous
import jax
import jax.numpy as jnp
from jax import lax
from jax.experimental import pallas as pl
from jax.experimental.pallas import tpu as pltpu

B = 8
H = 8
D = 128
BS = 16
HB = H * B
T = 512
PG = T // BS
TH = T * H

_MESH = pltpu.DeviceIdType.MESH
NEG_INF = -1e30


def _partial_body(qt_ref, bt_ref, lens_ref, k_ref, v_ref,
                  m_ref, l_ref, acc_ref):
    pb = pl.program_id(0)
    nb = bt_ref.shape[1]

    @pl.when(pb == 0)
    def _():
        m_ref[...] = jnp.full_like(m_ref, NEG_INF)
        l_ref[...] = jnp.zeros_like(l_ref)
        acc_ref[...] = jnp.zeros_like(acc_ref)

    my_x = lax.axis_index("x")
    n_local_pages = pl.num_programs(0) * PG
    base = my_x * n_local_pages + pb * PG
    bt = bt_ref[...]
    gid = base + lax.broadcasted_iota(jnp.int32, (PG, nb), 0)
    jio = lax.broadcasted_iota(jnp.int32, (PG, nb), 1)
    cols = []
    for b in range(B):
        hit = (bt[b:b + 1, :] == gid) & (jio < lens_ref[0, b])
        cols.append(jnp.sum(hit.astype(jnp.float32), axis=1, keepdims=True))
    wpg = jnp.concatenate(cols, axis=1)
    expand = (lax.broadcasted_iota(jnp.int32, (T, PG), 0) // BS
              == lax.broadcasted_iota(jnp.int32, (T, PG), 1))
    wt = lax.dot_general(
        expand.astype(jnp.float32), wpg,
        dimension_numbers=(((1,), (0,)), ((), ())),
        preferred_element_type=jnp.float32,
    )
    maskh = (lax.broadcasted_iota(jnp.int32, (H, HB), 1) // B
             == lax.broadcasted_iota(jnp.int32, (H, HB), 0)
             ).astype(jnp.float32)
    wt64 = jnp.concatenate([wt] * H, axis=1)
    w = (wt64[:, None, :] * maskh[None, :, :]).reshape(TH, HB)

    s = lax.dot_general(
        k_ref[...], qt_ref[...],
        dimension_numbers=(((1,), (0,)), ((), ())),
        preferred_element_type=jnp.float32,
    )

    m_old = m_ref[...]
    m_new = jnp.maximum(m_old, jnp.max(s, axis=0, keepdims=True))
    corr = jnp.exp(m_old - m_new)
    p = jnp.exp(s - m_new) * w
    l_ref[...] = l_ref[...] * corr + jnp.sum(p, axis=0, keepdims=True)

    pv = lax.dot_general(
        v_ref[...], p,
        dimension_numbers=(((0,), (0,)), ((), ())),
        preferred_element_type=jnp.float32,
    )
    acc_ref[...] = acc_ref[...] * corr + pv
    m_ref[...] = m_new


def _combine_body(m_ref, l_ref, acc_ref, out_ref,
                  m_r, l_r, acc_r, send_sems, recv_sems):
    my_x = lax.axis_index("x")
    my_y = lax.axis_index("y")
    my_z = lax.axis_index("z")
    peer = (1 - my_x, my_y, my_z)

    barrier_sem = pltpu.get_barrier_semaphore()
    pl.semaphore_signal(barrier_sem, inc=1, device_id=peer,
                        device_id_type=_MESH)
    pl.semaphore_wait(barrier_sem, 1)

    copies = []
    for i, (src, dst) in enumerate(((m_ref, m_r), (l_ref, l_r),
                                    (acc_ref, acc_r))):
        c = pltpu.make_async_remote_copy(
            src_ref=src, dst_ref=dst,
            send_sem=send_sems.at[i], recv_sem=recv_sems.at[i],
            device_id=peer, device_id_type=_MESH,
        )
        c.start()
        copies.append(c)
    for c in copies:
        c.wait()

    m_a, m_b = m_ref[...], m_r[...]
    m = jnp.maximum(m_a, m_b)
    ea = jnp.exp(m_a - m)
    eb = jnp.exp(m_b - m)
    l = l_ref[...] * ea + l_r[...] * eb
    o = (acc_ref[...] * ea + acc_r[...] * eb) / l
    for h in range(H):
        out_ref[:, 0, h, :] = o[:, h * B:(h + 1) * B].T


def kernel(Q, K, V, bt, lens):
    n_local = K.shape[0]
    n_tok = n_local * BS
    nb = bt.shape[1]
    assert n_tok % T == 0
    nsteps = n_tok // T

    q = Q.reshape(B, H, D).astype(jnp.float32) * (D ** -0.5)
    qt = q.transpose(2, 1, 0).reshape(D, HB)
    lens2 = lens.reshape(1, B)
    k2 = K.reshape(n_tok * H, D)
    v2 = V.reshape(n_tok * H, D)

    m, l, acc = pl.pallas_call(
        _partial_body,
        grid=(nsteps,),
        in_specs=[
            pl.BlockSpec((D, HB), lambda pb: (0, 0)),
            pl.BlockSpec((B, nb), lambda pb: (0, 0)),
            pl.BlockSpec(memory_space=pltpu.SMEM),
            pl.BlockSpec((TH, D), lambda pb: (pb, 0)),
            pl.BlockSpec((TH, D), lambda pb: (pb, 0)),
        ],
        out_specs=[
            pl.BlockSpec((1, HB), lambda pb: (0, 0)),
            pl.BlockSpec((1, HB), lambda pb: (0, 0)),
            pl.BlockSpec((D, HB), lambda pb: (0, 0)),
        ],
        out_shape=[
            jax.ShapeDtypeStruct((1, HB), jnp.float32),
            jax.ShapeDtypeStruct((1, HB), jnp.float32),
            jax.ShapeDtypeStruct((D, HB), jnp.float32),
        ],
    )(qt, bt, lens2, k2, v2)

    out = pl.pallas_call(
        _combine_body,
        in_specs=[
            pl.BlockSpec(memory_space=pltpu.VMEM),
            pl.BlockSpec(memory_space=pltpu.VMEM),
            pl.BlockSpec(memory_space=pltpu.VMEM),
        ],
        out_specs=pl.BlockSpec(memory_space=pltpu.VMEM),
        out_shape=jax.ShapeDtypeStruct((B, 1, H, D), jnp.float32),
        scratch_shapes=[
            pltpu.VMEM((1, HB), jnp.float32),
            pltpu.VMEM((1, HB), jnp.float32),
            pltpu.VMEM((D, HB), jnp.float32),
            pltpu.SemaphoreType.DMA((3,)),
            pltpu.SemaphoreType.DMA((3,)),
        ],
        compiler_params=pltpu.CompilerParams(collective_id=0),
    )(m, l, acc)
    return out


# device time: 40390 ns/iter; 2.2904x vs baseline; 1.0921x over previous
import jax
import jax.numpy as jnp
from jax import lax
from jax.experimental import pallas as pl
from jax.experimental.pallas import tpu as pltpu

B = 8
H = 8
D = 128
BS = 16
HB = H * B
T = 1024
PG = T // BS
TH = T * H

_MESH = pltpu.DeviceIdType.MESH
NEG_INF = -1e30


def _body(qt_ref, bt_ref, lens_ref, k_ref, v_ref, out_ref,
          m_ref, l_ref, acc_ref, m_r, l_r, acc_r, send_sems, recv_sems):
    pb = pl.program_id(0)
    nb = bt_ref.shape[1]

    @pl.when(pb == 0)
    def _():
        m_ref[...] = jnp.full_like(m_ref, NEG_INF)
        l_ref[...] = jnp.zeros_like(l_ref)
        acc_ref[...] = jnp.zeros_like(acc_ref)

    my_x = lax.axis_index("x")
    n_local_pages = pl.num_programs(0) * PG
    base = my_x * n_local_pages + pb * PG
    bt = bt_ref[...]
    gid = base + lax.broadcasted_iota(jnp.int32, (PG, nb), 0)
    jio = lax.broadcasted_iota(jnp.int32, (PG, nb), 1)
    cols = []
    for b in range(B):
        hit = (bt[b:b + 1, :] == gid) & (jio < lens_ref[0, b])
        cols.append(jnp.sum(hit.astype(jnp.float32), axis=1, keepdims=True))
    wpg = jnp.concatenate(cols, axis=1)
    expand = (lax.broadcasted_iota(jnp.int32, (T, PG), 0) // BS
              == lax.broadcasted_iota(jnp.int32, (T, PG), 1))
    wt = lax.dot_general(
        expand.astype(jnp.float32), wpg,
        dimension_numbers=(((1,), (0,)), ((), ())),
        preferred_element_type=jnp.float32,
    )
    maskh = (lax.broadcasted_iota(jnp.int32, (H, HB), 1) // B
             == lax.broadcasted_iota(jnp.int32, (H, HB), 0)
             ).astype(jnp.float32)
    wt64 = jnp.concatenate([wt] * H, axis=1)
    w = (wt64[:, None, :] * maskh[None, :, :]).reshape(TH, HB)

    s = lax.dot_general(
        k_ref[...].astype(jnp.bfloat16), qt_ref[...].astype(jnp.bfloat16),
        dimension_numbers=(((1,), (0,)), ((), ())),
        preferred_element_type=jnp.float32,
    )

    m_old = m_ref[...]
    m_new = jnp.maximum(m_old, jnp.max(s, axis=0, keepdims=True))
    corr = jnp.exp(m_old - m_new)
    p = jnp.exp(s - m_new) * w
    l_ref[...] = l_ref[...] * corr + jnp.sum(p, axis=0, keepdims=True)

    pv = lax.dot_general(
        v_ref[...].astype(jnp.bfloat16), p.astype(jnp.bfloat16),
        dimension_numbers=(((0,), (0,)), ((), ())),
        preferred_element_type=jnp.float32,
    )
    acc_ref[...] = acc_ref[...] * corr + pv
    m_ref[...] = m_new

    @pl.when(pb == pl.num_programs(0) - 1)
    def _():
        my_y = lax.axis_index("y")
        my_z = lax.axis_index("z")
        peer = (1 - my_x, my_y, my_z)

        barrier_sem = pltpu.get_barrier_semaphore()
        pl.semaphore_signal(barrier_sem, inc=1, device_id=peer,
                            device_id_type=_MESH)
        pl.semaphore_wait(barrier_sem, 1)

        copies = []
        for i, (src, dst) in enumerate(((m_ref, m_r), (l_ref, l_r),
                                        (acc_ref, acc_r))):
            c = pltpu.make_async_remote_copy(
                src_ref=src, dst_ref=dst,
                send_sem=send_sems.at[i], recv_sem=recv_sems.at[i],
                device_id=peer, device_id_type=_MESH,
            )
            c.start()
            copies.append(c)
        for c in copies:
            c.wait()

        m_a, m_b = m_ref[...], m_r[...]
        m = jnp.maximum(m_a, m_b)
        ea = jnp.exp(m_a - m)
        eb = jnp.exp(m_b - m)
        l = l_ref[...] * ea + l_r[...] * eb
        o = (acc_ref[...] * ea + acc_r[...] * eb) / l
        for h in range(H):
            out_ref[:, 0, h, :] = o[:, h * B:(h + 1) * B].T


def kernel(Q, K, V, bt, lens):
    n_local = K.shape[0]
    n_tok = n_local * BS
    nb = bt.shape[1]
    assert n_tok % T == 0
    nsteps = n_tok // T

    q = Q.reshape(B, H, D).astype(jnp.float32) * (D ** -0.5)
    qt = q.transpose(2, 1, 0).reshape(D, HB)
    lens2 = lens.reshape(1, B)
    k2 = K.reshape(n_tok * H, D)
    v2 = V.reshape(n_tok * H, D)

    return pl.pallas_call(
        _body,
        grid=(nsteps,),
        in_specs=[
            pl.BlockSpec((D, HB), lambda pb: (0, 0)),
            pl.BlockSpec((B, nb), lambda pb: (0, 0)),
            pl.BlockSpec(memory_space=pltpu.SMEM),
            pl.BlockSpec((TH, D), lambda pb: (pb, 0)),
            pl.BlockSpec((TH, D), lambda pb: (pb, 0)),
        ],
        out_specs=pl.BlockSpec((B, 1, H, D), lambda pb: (0, 0, 0, 0)),
        out_shape=jax.ShapeDtypeStruct((B, 1, H, D), jnp.float32),
        scratch_shapes=[
            pltpu.VMEM((1, HB), jnp.float32),
            pltpu.VMEM((1, HB), jnp.float32),
            pltpu.VMEM((D, HB), jnp.float32),
            pltpu.VMEM((1, HB), jnp.float32),
            pltpu.VMEM((1, HB), jnp.float32),
            pltpu.VMEM((D, HB), jnp.float32),
            pltpu.SemaphoreType.DMA((3,)),
            pltpu.SemaphoreType.DMA((3,)),
        ],
        compiler_params=pltpu.CompilerParams(collective_id=0),
    )(qt, bt, lens2, k2, v2)
